# baseline (speedup 1.0000x reference)
import functools

import jax
import jax.numpy as jnp
from jax import lax
from jax.experimental import pallas as pl
from jax.experimental.pallas import tpu as pltpu
from jax.experimental.pallas import tpu_sc as plsc

N_NODES_K = 10000
N_EDGES_K = 320000
NC = 2
NS = 16
NW = NC * NS
LANES = 16
EPW = N_EDGES_K // NW
HPAD = 10240
CPT = HPAD // NS


def _hist_body(recv_hbm, parts_hbm, idx_v, hist_v, slab_v, shared):
  c = lax.axis_index("c")
  s = lax.axis_index("s")
  wid = s * NC + c

  pltpu.sync_copy(recv_hbm.at[pl.ds(wid * EPW, EPW)], idx_v)

  def zero(i, carry):
    hist_v[pl.ds(i * LANES, LANES)] = jnp.zeros((LANES,), jnp.float32)
    return carry

  lax.fori_loop(0, HPAD // LANES, zero, 0)

  def body(i, carry):
    v = idx_v[pl.ds(i * LANES, LANES)]
    cnt, last = plsc.scan_count(v)
    plsc.addupdate_scatter(hist_v, [v], cnt.astype(jnp.float32), mask=last)
    return carry

  lax.fori_loop(0, EPW // LANES, body, 0)

  pltpu.sync_copy(hist_v, shared.at[s])
  plsc.subcore_barrier()
  pltpu.sync_copy(shared.at[:, pl.ds(s * CPT, CPT)], slab_v)

  def col(i, carry):
    sl = pl.ds(i * LANES, LANES)
    acc = slab_v[0, sl]
    for r in range(1, NS):
      acc = acc + slab_v[r, sl]
    hist_v[sl] = acc
    return carry

  lax.fori_loop(0, CPT // LANES, col, 0)

  pltpu.sync_copy(hist_v.at[pl.ds(0, CPT)],
                  parts_hbm.at[pl.ds(c * HPAD + s * CPT, CPT)])


_hist = pl.kernel(
    _hist_body,
    out_type=jax.ShapeDtypeStruct((NC * HPAD,), jnp.float32),
    mesh=plsc.VectorSubcoreMesh(
        core_axis_name="c", subcore_axis_name="s", num_cores=NC,
        num_subcores=NS),
    scratch_types=[
        pltpu.VMEM((EPW,), jnp.int32),
        pltpu.VMEM((HPAD,), jnp.float32),
        pltpu.VMEM((NS, CPT), jnp.float32),
        pltpu.VMEM_SHARED((NS, HPAD), jnp.float32),
    ],
    compiler_params=pltpu.CompilerParams(needs_layout_passes=False),
)


def _merge_body(parts_ref, out_ref):
  out_ref[...] = jnp.sum(parts_ref[...], axis=0, keepdims=True)


_merge = pl.pallas_call(
    _merge_body,
    out_shape=jax.ShapeDtypeStruct((1, HPAD), jnp.float32),
)


@jax.jit
def kernel(x, edge_index):
  n = x.shape[0]
  parts = _hist(edge_index[1].astype(jnp.int32))
  merged = _merge(parts.reshape(NC, HPAD))
  return merged[0, :n].reshape(n, 1)


# speedup vs baseline: 1.1023x; 1.1023x over previous
import functools

import jax
import jax.numpy as jnp
from jax import lax
from jax.experimental import pallas as pl
from jax.experimental.pallas import tpu as pltpu
from jax.experimental.pallas import tpu_sc as plsc

N_NODES_K = 10000
N_EDGES_K = 320000
NC = 2
NS = 16
NW = NC * NS
LANES = 16
EPW = N_EDGES_K // NW
HPAD = 10240


def _hist_body(recv_hbm, zeros_hbm, parts_hbm, idx_v, hist_v, cw_v, sem_i,
               sem_z):
  c = lax.axis_index("c")
  s = lax.axis_index("s")
  wid = s * NC + c

  idx_dma = pltpu.make_async_copy(recv_hbm.at[pl.ds(wid * EPW, EPW)], idx_v,
                                  sem_i)
  zero_dma = pltpu.make_async_copy(zeros_hbm, hist_v, sem_z)
  idx_dma.start()
  zero_dma.start()
  idx_dma.wait()

  def pass1(i, carry):
    sl = pl.ds(i * LANES, LANES)
    v = idx_v[sl]
    cnt, last = plsc.scan_count(v)
    cw_v[sl] = jnp.where(last, cnt.astype(jnp.float32), 0.0)
    return carry

  lax.fori_loop(0, EPW // LANES, pass1, 0)

  zero_dma.wait()

  def pass2(i, carry):
    sl = pl.ds(i * LANES, LANES)
    v = idx_v[sl]
    mv = cw_v[sl]
    plsc.addupdate_scatter(hist_v, [v], mv, mask=mv > 0.5)
    return carry

  lax.fori_loop(0, EPW // LANES, pass2, 0)

  pltpu.sync_copy(hist_v, parts_hbm.at[wid])


_hist = pl.kernel(
    _hist_body,
    out_type=jax.ShapeDtypeStruct((NW, HPAD), jnp.float32),
    mesh=plsc.VectorSubcoreMesh(
        core_axis_name="c", subcore_axis_name="s", num_cores=NC,
        num_subcores=NS),
    scratch_types=[
        pltpu.VMEM((EPW,), jnp.int32),
        pltpu.VMEM((HPAD,), jnp.float32),
        pltpu.VMEM((EPW,), jnp.float32),
        pltpu.SemaphoreType.DMA,
        pltpu.SemaphoreType.DMA,
    ],
    compiler_params=pltpu.CompilerParams(needs_layout_passes=False),
)


def _merge_body(parts_ref, out_ref):
  out_ref[...] = jnp.sum(parts_ref[...], axis=0, keepdims=True)


_merge = pl.pallas_call(
    _merge_body,
    out_shape=jax.ShapeDtypeStruct((1, HPAD), jnp.float32),
)


@jax.jit
def kernel(x, edge_index):
  n = x.shape[0]
  zeros = jnp.zeros((HPAD,), jnp.float32)
  parts = _hist(edge_index[1].astype(jnp.int32), zeros)
  merged = _merge(parts)
  return merged[0, :n].reshape(n, 1)


# speedup vs baseline: 1.4644x; 1.3286x over previous
import functools

import jax
import jax.numpy as jnp
from jax import lax
from jax.experimental import pallas as pl
from jax.experimental.pallas import tpu as pltpu
from jax.experimental.pallas import tpu_sc as plsc

N_NODES_K = 10000
N_EDGES_K = 320000
NC = 2
NS = 16
NW = NC * NS
LANES = 16
EPW = N_EDGES_K // NW
HPAD = 10240


def _hist_body(recv_hbm, zeros_hbm, parts_hbm, idx_v, hist_v, cw_v, sem_i,
               sem_z):
  c = lax.axis_index("c")
  s = lax.axis_index("s")
  wid = s * NC + c

  idx_dma = pltpu.make_async_copy(
      recv_hbm.at[pl.ds(N_EDGES_K + wid * EPW, EPW)], idx_v, sem_i)
  zero_dma = pltpu.make_async_copy(zeros_hbm, hist_v, sem_z)
  idx_dma.start()
  zero_dma.start()
  idx_dma.wait()

  def p1_step(i):
    sl = pl.ds(i * LANES, LANES)
    v = idx_v[sl]
    cnt, last = plsc.scan_count(v)
    cw_v[sl] = jnp.where(last, cnt.astype(jnp.float32), 0.0)

  def pass1(i, carry):
    p1_step(2 * i)
    p1_step(2 * i + 1)
    return carry

  lax.fori_loop(0, EPW // (2 * LANES), pass1, 0)

  def p1_tail(i, carry):
    p1_step((EPW // (2 * LANES)) * 2 + i)
    return carry

  lax.fori_loop(0, (EPW // LANES) - (EPW // (2 * LANES)) * 2, p1_tail, 0)

  zero_dma.wait()

  def p2_step(i):
    sl = pl.ds(i * LANES, LANES)
    v = idx_v[sl]
    mv = cw_v[sl]
    plsc.addupdate_scatter(hist_v, [v], mv, mask=mv > 0.5)

  def pass2(i, carry):
    for j in range(4):
      p2_step(4 * i + j)
    return carry

  lax.fori_loop(0, EPW // (4 * LANES), pass2, 0)

  def p2_tail(i, carry):
    p2_step((EPW // (4 * LANES)) * 4 + i)
    return carry

  lax.fori_loop(0, (EPW // LANES) - (EPW // (4 * LANES)) * 4, p2_tail, 0)

  pltpu.sync_copy(hist_v, parts_hbm.at[wid])


_hist = pl.kernel(
    _hist_body,
    out_type=jax.ShapeDtypeStruct((NW, HPAD), jnp.float32),
    mesh=plsc.VectorSubcoreMesh(
        core_axis_name="c", subcore_axis_name="s", num_cores=NC,
        num_subcores=NS),
    scratch_types=[
        pltpu.VMEM((EPW,), jnp.int32),
        pltpu.VMEM((HPAD,), jnp.float32),
        pltpu.VMEM((EPW,), jnp.float32),
        pltpu.SemaphoreType.DMA,
        pltpu.SemaphoreType.DMA,
    ],
    compiler_params=pltpu.CompilerParams(needs_layout_passes=False),
)


def _merge_body(parts_ref, out_ref):
  out_ref[...] = jnp.sum(parts_ref[...], axis=0, keepdims=True)


_merge = pl.pallas_call(
    _merge_body,
    out_shape=jax.ShapeDtypeStruct((1, HPAD), jnp.float32),
)


@jax.jit
def kernel(x, edge_index):
  n = x.shape[0]
  zeros = jnp.zeros((HPAD,), jnp.float32)
  parts = _hist(edge_index.astype(jnp.int32).reshape(-1), zeros)
  merged = _merge(parts)
  return merged[0, :n].reshape(n, 1)


# speedup vs baseline: 1.4815x; 1.0117x over previous
import functools

import jax
import jax.numpy as jnp
from jax import lax
from jax.experimental import pallas as pl
from jax.experimental.pallas import tpu as pltpu
from jax.experimental.pallas import tpu_sc as plsc

N_NODES_K = 10000
N_EDGES_K = 320000
NC = 2
NS = 16
NW = NC * NS
LANES = 16
EPW = N_EDGES_K // NW
HPAD = 10240


def _hist_body(recv_hbm, zeros_hbm, parts_hbm, idx_v, hist_v, cw_v, sem_i,
               sem_z):
  c = lax.axis_index("c")
  s = lax.axis_index("s")
  wid = s * NC + c

  idx_dma = pltpu.make_async_copy(
      recv_hbm.at[pl.ds(N_EDGES_K + wid * EPW, EPW)], idx_v, sem_i)
  zero_dma = pltpu.make_async_copy(zeros_hbm, hist_v, sem_z)
  idx_dma.start()
  zero_dma.start()
  idx_dma.wait()

  def p1_step(i):
    sl = pl.ds(i * LANES, LANES)
    v = idx_v[sl]
    cnt, last = plsc.scan_count(v)
    cw_v[sl] = jnp.where(last, cnt.astype(jnp.float32), 0.0)

  def pass1(i, carry):
    p1_step(3 * i)
    p1_step(3 * i + 1)
    p1_step(3 * i + 2)
    return carry

  lax.fori_loop(0, EPW // (3 * LANES), pass1, 0)

  def p1_tail(i, carry):
    p1_step((EPW // (3 * LANES)) * 3 + i)
    return carry

  lax.fori_loop(0, (EPW // LANES) - (EPW // (3 * LANES)) * 3, p1_tail, 0)

  zero_dma.wait()

  def p2_step(i):
    sl = pl.ds(i * LANES, LANES)
    v = idx_v[sl]
    mv = cw_v[sl]
    plsc.addupdate_scatter(hist_v, [v], mv, mask=mv > 0.5)

  def pass2(i, carry):
    for j in range(8):
      p2_step(8 * i + j)
    return carry

  lax.fori_loop(0, EPW // (8 * LANES), pass2, 0)

  def p2_tail(i, carry):
    p2_step((EPW // (8 * LANES)) * 8 + i)
    return carry

  lax.fori_loop(0, (EPW // LANES) - (EPW // (8 * LANES)) * 8, p2_tail, 0)

  pltpu.sync_copy(hist_v, parts_hbm.at[wid])


_hist = pl.kernel(
    _hist_body,
    out_type=jax.ShapeDtypeStruct((NW, HPAD), jnp.float32),
    mesh=plsc.VectorSubcoreMesh(
        core_axis_name="c", subcore_axis_name="s", num_cores=NC,
        num_subcores=NS),
    scratch_types=[
        pltpu.VMEM((EPW,), jnp.int32),
        pltpu.VMEM((HPAD,), jnp.float32),
        pltpu.VMEM((EPW,), jnp.float32),
        pltpu.SemaphoreType.DMA,
        pltpu.SemaphoreType.DMA,
    ],
    compiler_params=pltpu.CompilerParams(needs_layout_passes=False),
)


def _merge_body(parts_ref, out_ref):
  out_ref[...] = jnp.sum(parts_ref[...], axis=0, keepdims=True)


_merge = pl.pallas_call(
    _merge_body,
    out_shape=jax.ShapeDtypeStruct((1, HPAD), jnp.float32),
)


@jax.jit
def kernel(x, edge_index):
  n = x.shape[0]
  zeros = jnp.zeros((HPAD,), jnp.float32)
  parts = _hist(edge_index.astype(jnp.int32).reshape(-1), zeros)
  merged = _merge(parts)
  return merged[0, :n].reshape(n, 1)
